# baseline (speedup 1.0000x reference)
import functools

import jax
import jax.numpy as jnp
from jax.experimental import pallas as pl
from jax.experimental.pallas import tpu as pltpu

F32 = jnp.float32

_S2_KH = {(1, -1): 0, (0, 0): 1, (1, 0): 2, (0, 1): 3}
_S2_DI = {-1: (1,), 0: (0, 1), 1: (0,)}
_CT_K = {(0, 0): 1, (0, -1): 3, (1, 1): 0, (1, 0): 2}


def _s2_weight_mats(w):
    O, C = w.shape[0], w.shape[1]
    out = []
    for di in (-1, 0, 1):
        a_list = [a for a in (0, 1) if (a, di) in _S2_KH]
        ch0 = 2 * C if di == -1 else 0
        for dj in (-1, 0, 1):
            m = jnp.zeros((len(a_list), 2, C, O), F32)
            for ai, a in enumerate(a_list):
                for b in (0, 1):
                    if (b, dj) in _S2_KH:
                        kh, kw = _S2_KH[(a, di)], _S2_KH[(b, dj)]
                        m = m.at[ai, b].set(w[:, :, kh, kw].T)
            out.append((di, dj, ch0, m.reshape(len(a_list) * 2 * C, O)))
    return out


def _ct_weight_mats(w):
    Cin, Cout = w.shape[0], w.shape[1]
    out = []
    for di in (-1, 0, 1):
        for dj in (-1, 0, 1):
            m = jnp.zeros((Cin, 2, 2, Cout), F32)
            used = False
            for a in (0, 1):
                if (a, di) not in _CT_K:
                    continue
                for b in (0, 1):
                    if (b, dj) not in _CT_K:
                        continue
                    m = m.at[:, a, b].set(w[:, :, _CT_K[(a, di)], _CT_K[(b, dj)]])
                    used = True
            if used:
                out.append((di, dj, 0, m.reshape(Cin, 4 * Cout)))
    return out


def _ct_weight_mats_asplit(w):
    Cin, Cout = w.shape[0], w.shape[1]
    out = []
    for a in (0, 1):
        for di in (-1, 0, 1):
            if (a, di) not in _CT_K:
                continue
            for dj in (-1, 0, 1):
                m = jnp.zeros((Cin, 2, Cout), F32)
                for b in (0, 1):
                    if (b, dj) in _CT_K:
                        m = m.at[:, b].set(
                            w[:, :, _CT_K[(a, di)], _CT_K[(b, dj)]])
                out.append((di, dj, a, m.reshape(Cin, 2 * Cout)))
    return out


def _s2d_pad_flat(x):
    n, h, w, c = x.shape
    h2, w2 = h // 2, w // 2
    t = x.reshape(n, h2, 2, w2, 2, c).transpose(0, 1, 3, 2, 4, 5)
    t = t.reshape(n, h2, w2, 4 * c)
    t = jnp.pad(t, ((0, 0), (2, 2), (1, 1), (0, 0)))
    return t.reshape(n, (h2 + 4) * (w2 + 2), 4 * c)


def _pad_flat(x):
    n, h, w, c = x.shape
    t = jnp.pad(x, ((0, 0), (2, 2), (1, 1), (0, 0)))
    return t.reshape(n, (h + 4) * (w + 2), c)


def _unflat(y, h, w):
    n = y.shape[0]
    return y.reshape(n, h, w + 2, y.shape[-1])[:, :, 1:w + 1, :]


def _act(acc, act):
    if act == "relu":
        return jnp.maximum(acc, 0.0)
    if act == "sigmoid":
        return jax.nn.sigmoid(acc)
    return acc


def _conv_body(x_ref, *rest, taps, m_rows, p, act):
    out_ref = rest[-1]
    b_ref = rest[-2]
    w_refs = rest[:-2]
    acc = jnp.broadcast_to(b_ref[...][None, :], (m_rows, b_ref.shape[0]))
    for (di, dj, ch0, kdim), w_ref in zip(taps, w_refs):
        start = (2 + di) * p + dj
        xs = x_ref[0, start:start + m_rows, ch0:ch0 + kdim]
        acc = acc + jnp.dot(xs, w_ref[...], preferred_element_type=F32)
    out_ref[0] = _act(acc, act).astype(out_ref.dtype)


def _conv_body_asplit(x_ref, *rest, taps, m_rows, p, act):
    out_ref = rest[-1]
    b_ref = rest[-2]
    w_refs = rest[:-2]
    half = b_ref.shape[0] // 2
    accs = [jnp.broadcast_to(b_ref[a * half:(a + 1) * half][None, :],
                             (m_rows, half)) for a in (0, 1)]
    for (di, dj, ch0, kdim, a), w_ref in zip(taps, w_refs):
        start = (2 + di) * p + dj
        xs = x_ref[0, start:start + m_rows, ch0:ch0 + kdim]
        accs[a] = accs[a] + jnp.dot(xs, w_ref[...],
                                    preferred_element_type=F32)
    acc = jnp.concatenate([accs[0], accs[1]], axis=1)
    out_ref[0] = _act(acc, act).astype(out_ref.dtype)


def _conv_body_kmerge(x_ref, w_ref, b_ref, out_ref, scratch, *, taps, m_rows,
                      p, act):
    off = 0
    for di, dj, ch0, kdim in taps:
        start = (2 + di) * p + dj
        scratch[:, off:off + kdim] = x_ref[0, start:start + m_rows,
                                           ch0:ch0 + kdim]
        off += kdim
    acc = (jnp.dot(scratch[...], w_ref[...], preferred_element_type=F32)
           + b_ref[...][None, :])
    out_ref[0] = _act(acc, act).astype(out_ref.dtype)


def _conv_layer(x_flat, mats, bias, h_out, p, act, out_dtype=jnp.bfloat16,
                mode="shift"):
    n, f, cin_tot = x_flat.shape
    m_rows = h_out * p
    bf16 = jnp.bfloat16
    if mode == "asplit":
        cout = mats[0][3].shape[1] * 2
        taps = [(di, dj, 0, wm.shape[0], a) for di, dj, a, wm in mats]
        body = functools.partial(_conv_body_asplit, taps=taps, m_rows=m_rows,
                                 p=p, act=act)
    else:
        cout = mats[0][3].shape[1]
        taps = [(di, dj, ch0, wm.shape[0]) for di, dj, ch0, wm in mats]
        if mode == "kmerge":
            body = functools.partial(_conv_body_kmerge, taps=taps,
                                     m_rows=m_rows, p=p, act=act)
        else:
            body = functools.partial(_conv_body, taps=taps, m_rows=m_rows,
                                     p=p, act=act)
    in_specs = [pl.BlockSpec((1, f, cin_tot), lambda i: (i, 0, 0))]
    if mode == "kmerge":
        wcat = jnp.concatenate([wm for _, _, _, wm in mats],
                               axis=0).astype(bf16)
        in_specs += [pl.BlockSpec(wcat.shape, lambda i: (0, 0))]
        weights = [wcat]
        scratch_shapes = [pltpu.VMEM((m_rows, wcat.shape[0]), bf16)]
    else:
        in_specs += [pl.BlockSpec(wm.shape[-2:], lambda i: (0, 0))
                     for wm in [m[3] for m in mats]]
        weights = [m[3].astype(bf16) for m in mats]
        scratch_shapes = []
    in_specs += [pl.BlockSpec((cout,), lambda i: (0,))]
    return pl.pallas_call(
        body,
        grid=(n,),
        in_specs=in_specs,
        out_specs=pl.BlockSpec((1, m_rows, cout), lambda i: (i, 0, 0)),
        out_shape=jax.ShapeDtypeStruct((n, m_rows, cout), out_dtype),
        scratch_shapes=scratch_shapes,
    )(x_flat, *weights, bias)


def _vq_body(z_ref, emb_ref, zq_ref, loss_ref):
    z = z_ref[0]
    e = emb_ref[...]
    zf = z.astype(F32)
    ef = e.astype(F32)
    g = jax.lax.dot_general(z, e, (((1,), (1,)), ((), ())),
                            preferred_element_type=F32)
    zn = jnp.sum(zf * zf, axis=1, keepdims=True)
    en = jnp.sum(ef * ef, axis=1)
    d = (zn + en[None, :]) - 2.0 * g
    dmin = jnp.min(d, axis=1, keepdims=True)
    k = e.shape[0]
    iot = jax.lax.broadcasted_iota(jnp.int32, d.shape, 1)
    idx = jnp.min(jnp.where(d == dmin, iot, k), axis=1, keepdims=True)
    onehot = (iot == idx).astype(z.dtype)
    zq = jnp.dot(onehot, e, preferred_element_type=F32)
    zq_ref[0] = zq.astype(zq_ref.dtype)
    loss_ref[0] = jnp.broadcast_to(jnp.sum((zq - zf) ** 2) / 128.0, (1, 128))


def _vq(zf, emb, rows_per_chunk):
    r, dch = zf.shape
    k = emb.shape[0]
    nchunks = r // rows_per_chunk
    assert nchunks * rows_per_chunk == r
    zq, partial = pl.pallas_call(
        _vq_body,
        grid=(nchunks,),
        in_specs=[pl.BlockSpec((1, rows_per_chunk, dch), lambda i: (i, 0, 0)),
                  pl.BlockSpec((k, dch), lambda i: (0, 0))],
        out_specs=[pl.BlockSpec((1, rows_per_chunk, dch), lambda i: (i, 0, 0)),
                   pl.BlockSpec((1, 1, 128), lambda i: (i, 0, 0))],
        out_shape=[jax.ShapeDtypeStruct((nchunks, rows_per_chunk, dch),
                                        jnp.bfloat16),
                   jax.ShapeDtypeStruct((nchunks, 1, 128), F32)],
    )(zf.reshape(nchunks, rows_per_chunk, dch), emb.astype(jnp.bfloat16))
    return zq.reshape(r, dch), jnp.sum(partial)


def _d2s(y4, h, w, cout):
    n = y4.shape[0]
    t = y4.reshape(n, h, w + 2, 2, 2, cout)[:, :, 1:w + 1]
    t = t.transpose(0, 1, 3, 2, 4, 5)
    return t.reshape(n, 2 * h, 2 * w, cout)


def kernel(x, w1, b1, w2, b2, w3, b3, emb_top, emb_bottom, wt, bt, wd1, bd1,
           wd2, bd2):
    n = x.shape[0]
    xh = jnp.transpose(x, (0, 2, 3, 1)).astype(jnp.bfloat16)

    x1f = _s2d_pad_flat(xh)
    m1 = 112 * 114
    mats1 = _s2_weight_mats(w1)
    xcat = jnp.concatenate(
        [x1f[:, (2 + di) * 114 + dj:(2 + di) * 114 + dj + m1, ch0:ch0 + wm.shape[0]]
         for di, dj, ch0, wm in mats1], axis=-1)
    wcat1 = jnp.concatenate([wm for _, _, _, wm in mats1], axis=0)
    y1 = _conv_layer(xcat, [(-2, 0, 0, wcat1)], b1,
                     112, 114, "relu")
    a1 = _unflat(y1, 112, 112)
    y2 = _conv_layer(_s2d_pad_flat(a1), _s2_weight_mats(w2), b2,
                     56, 58, "relu")
    zb = _unflat(y2, 56, 56)
    y3 = _conv_layer(_s2d_pad_flat(zb), _s2_weight_mats(w3), b3,
                     28, 30, "relu")
    zt = _unflat(y3, 28, 28)

    zt_q, ss_top = _vq(zt.reshape(n * 28 * 28, 128), emb_top, 3136)
    loss_top = 1.5 * ss_top / (n * 28 * 28 * 128)
    zt_q = zt_q.reshape(n, 28, 28, 128)

    y4 = _conv_layer(_pad_flat(zt_q), _ct_weight_mats_asplit(wt),
                     jnp.tile(bt, 4), 28, 30, "relu", mode="asplit")
    zt_up = _d2s(y4, 28, 28, 128)

    zb_q, ss_bot = _vq(zb.reshape(n * 56 * 56, 128), emb_bottom, 3136)
    loss_bottom = 1.5 * ss_bot / (n * 56 * 56 * 128)
    zb_q = zb_q.reshape(n, 56, 56, 128)

    zc = jnp.concatenate([zb_q, zt_up], axis=-1)
    y5 = _conv_layer(_pad_flat(zc), _ct_weight_mats_asplit(wd1),
                     jnp.tile(bd1, 4), 56, 58, "relu", mode="asplit")
    h1 = _d2s(y5, 56, 56, 64)

    y6 = _conv_layer(_pad_flat(h1), _ct_weight_mats(wd2),
                     jnp.tile(bd2, 4), 112, 114, "sigmoid", out_dtype=F32)
    x_hat = _d2s(y6, 112, 112, 3)
    x_hat = jnp.transpose(x_hat, (0, 3, 1, 2))

    return x_hat, loss_top + loss_bottom


# speedup vs baseline: 1.2778x; 1.2778x over previous
import functools

import jax
import jax.numpy as jnp
from jax.experimental import pallas as pl
from jax.experimental.pallas import tpu as pltpu

F32 = jnp.float32

_S2_KH = {(1, -1): 0, (0, 0): 1, (1, 0): 2, (0, 1): 3}
_S2_DI = {-1: (1,), 0: (0, 1), 1: (0,)}
_CT_K = {(0, 0): 1, (0, -1): 3, (1, 1): 0, (1, 0): 2}


def _s2_weight_mats(w):
    O, C = w.shape[0], w.shape[1]
    out = []
    for di in (-1, 0, 1):
        a_list = [a for a in (0, 1) if (a, di) in _S2_KH]
        ch0 = 2 * C if di == -1 else 0
        for dj in (-1, 0, 1):
            m = jnp.zeros((len(a_list), 2, C, O), F32)
            for ai, a in enumerate(a_list):
                for b in (0, 1):
                    if (b, dj) in _S2_KH:
                        kh, kw = _S2_KH[(a, di)], _S2_KH[(b, dj)]
                        m = m.at[ai, b].set(w[:, :, kh, kw].T)
            out.append((di, dj, ch0, m.reshape(len(a_list) * 2 * C, O)))
    return out


def _ct_weight_mats(w):
    Cin, Cout = w.shape[0], w.shape[1]
    out = []
    for di in (-1, 0, 1):
        for dj in (-1, 0, 1):
            m = jnp.zeros((Cin, 2, 2, Cout), F32)
            used = False
            for a in (0, 1):
                if (a, di) not in _CT_K:
                    continue
                for b in (0, 1):
                    if (b, dj) not in _CT_K:
                        continue
                    m = m.at[:, a, b].set(w[:, :, _CT_K[(a, di)], _CT_K[(b, dj)]])
                    used = True
            if used:
                out.append((di, dj, 0, m.reshape(Cin, 4 * Cout)))
    return out


def _ct_weight_mats_asplit(w):
    Cin, Cout = w.shape[0], w.shape[1]
    out = []
    for a in (0, 1):
        for di in (-1, 0, 1):
            if (a, di) not in _CT_K:
                continue
            for dj in (-1, 0, 1):
                m = jnp.zeros((Cin, 2, Cout), F32)
                for b in (0, 1):
                    if (b, dj) in _CT_K:
                        m = m.at[:, b].set(
                            w[:, :, _CT_K[(a, di)], _CT_K[(b, dj)]])
                out.append((di, dj, a, m.reshape(Cin, 2 * Cout)))
    return out


def _s2d_pad_flat(x):
    n, h, w, c = x.shape
    h2, w2 = h // 2, w // 2
    t = x.reshape(n, h2, 2, w2, 2, c).transpose(0, 1, 3, 2, 4, 5)
    t = t.reshape(n, h2, w2, 4 * c)
    t = jnp.pad(t, ((0, 0), (2, 2), (1, 1), (0, 0)))
    return t.reshape(n, (h2 + 4) * (w2 + 2), 4 * c)


def _pad_flat(x):
    n, h, w, c = x.shape
    t = jnp.pad(x, ((0, 0), (2, 2), (1, 1), (0, 0)))
    return t.reshape(n, (h + 4) * (w + 2), c)


def _unflat(y, h, w):
    n = y.shape[0]
    return y.reshape(n, h, w + 2, y.shape[-1])[:, :, 1:w + 1, :]


def _act(acc, act):
    if act == "relu":
        return jnp.maximum(acc, 0.0)
    if act == "sigmoid":
        return jax.nn.sigmoid(acc)
    return acc


def _conv_body(x_ref, *rest, taps, m_rows, p, act):
    out_ref = rest[-1]
    b_ref = rest[-2]
    w_refs = rest[:-2]
    acc = jnp.broadcast_to(b_ref[...][None, :], (m_rows, b_ref.shape[0]))
    for (di, dj, ch0, kdim), w_ref in zip(taps, w_refs):
        start = (2 + di) * p + dj
        xs = x_ref[0, start:start + m_rows, ch0:ch0 + kdim]
        acc = acc + jnp.dot(xs, w_ref[...], preferred_element_type=F32)
    out_ref[0] = _act(acc, act).astype(out_ref.dtype)


def _conv_body_asplit(x_ref, *rest, taps, m_rows, p, act):
    out_ref = rest[-1]
    b_ref = rest[-2]
    w_refs = rest[:-2]
    half = b_ref.shape[0] // 2
    accs = [jnp.broadcast_to(b_ref[a * half:(a + 1) * half][None, :],
                             (m_rows, half)) for a in (0, 1)]
    for (di, dj, ch0, kdim, a), w_ref in zip(taps, w_refs):
        start = (2 + di) * p + dj
        xs = x_ref[0, start:start + m_rows, ch0:ch0 + kdim]
        accs[a] = accs[a] + jnp.dot(xs, w_ref[...],
                                    preferred_element_type=F32)
    acc = jnp.concatenate([accs[0], accs[1]], axis=1)
    out_ref[0] = _act(acc, act).astype(out_ref.dtype)


def _conv_body_kmerge(x_ref, w_ref, b_ref, out_ref, scratch, *, taps, m_rows,
                      p, act):
    off = 0
    for di, dj, ch0, kdim in taps:
        start = (2 + di) * p + dj
        scratch[:, off:off + kdim] = x_ref[0, start:start + m_rows,
                                           ch0:ch0 + kdim]
        off += kdim
    acc = (jnp.dot(scratch[...], w_ref[...], preferred_element_type=F32)
           + b_ref[...][None, :])
    out_ref[0] = _act(acc, act).astype(out_ref.dtype)


def _conv_layer(x_flat, mats, bias, h_out, p, act, out_dtype=jnp.bfloat16,
                mode="shift"):
    n, f, cin_tot = x_flat.shape
    m_rows = h_out * p
    bf16 = jnp.bfloat16
    if mode == "asplit":
        cout = mats[0][3].shape[1] * 2
        taps = [(di, dj, 0, wm.shape[0], a) for di, dj, a, wm in mats]
        body = functools.partial(_conv_body_asplit, taps=taps, m_rows=m_rows,
                                 p=p, act=act)
    else:
        cout = mats[0][3].shape[1]
        taps = [(di, dj, ch0, wm.shape[0]) for di, dj, ch0, wm in mats]
        if mode == "kmerge":
            body = functools.partial(_conv_body_kmerge, taps=taps,
                                     m_rows=m_rows, p=p, act=act)
        else:
            body = functools.partial(_conv_body, taps=taps, m_rows=m_rows,
                                     p=p, act=act)
    in_specs = [pl.BlockSpec((1, f, cin_tot), lambda i: (i, 0, 0))]
    if mode == "kmerge":
        wcat = jnp.concatenate([wm for _, _, _, wm in mats],
                               axis=0).astype(bf16)
        in_specs += [pl.BlockSpec(wcat.shape, lambda i: (0, 0))]
        weights = [wcat]
        scratch_shapes = [pltpu.VMEM((m_rows, wcat.shape[0]), bf16)]
    else:
        in_specs += [pl.BlockSpec(wm.shape[-2:], lambda i: (0, 0))
                     for wm in [m[3] for m in mats]]
        weights = [m[3].astype(bf16) for m in mats]
        scratch_shapes = []
    in_specs += [pl.BlockSpec((cout,), lambda i: (0,))]
    return pl.pallas_call(
        body,
        grid=(n,),
        in_specs=in_specs,
        out_specs=pl.BlockSpec((1, m_rows, cout), lambda i: (i, 0, 0)),
        out_shape=jax.ShapeDtypeStruct((n, m_rows, cout), out_dtype),
        scratch_shapes=scratch_shapes,
    )(x_flat, *weights, bias)


def _vq_body(z_ref, emb_ref, zq_ref, loss_ref):
    z = z_ref[0]
    e = emb_ref[...]
    zf = z.astype(F32)
    ef = e.astype(F32)
    g = jax.lax.dot_general(z, e, (((1,), (1,)), ((), ())),
                            preferred_element_type=F32)
    zn = jnp.sum(zf * zf, axis=1, keepdims=True)
    en = jnp.sum(ef * ef, axis=1)
    d = (zn + en[None, :]) - 2.0 * g
    dmin = jnp.min(d, axis=1, keepdims=True)
    k = e.shape[0]
    iot = jax.lax.broadcasted_iota(jnp.int32, d.shape, 1)
    idx = jnp.min(jnp.where(d == dmin, iot, k), axis=1, keepdims=True)
    onehot = (iot == idx).astype(z.dtype)
    zq = jnp.dot(onehot, e, preferred_element_type=F32)
    zq_ref[0] = zq.astype(zq_ref.dtype)
    loss_ref[0] = jnp.broadcast_to(jnp.sum((zq - zf) ** 2) / 128.0, (1, 128))


def _vq(zf, emb, rows_per_chunk):
    r, dch = zf.shape
    k = emb.shape[0]
    nchunks = r // rows_per_chunk
    assert nchunks * rows_per_chunk == r
    zq, partial = pl.pallas_call(
        _vq_body,
        grid=(nchunks,),
        in_specs=[pl.BlockSpec((1, rows_per_chunk, dch), lambda i: (i, 0, 0)),
                  pl.BlockSpec((k, dch), lambda i: (0, 0))],
        out_specs=[pl.BlockSpec((1, rows_per_chunk, dch), lambda i: (i, 0, 0)),
                   pl.BlockSpec((1, 1, 128), lambda i: (i, 0, 0))],
        out_shape=[jax.ShapeDtypeStruct((nchunks, rows_per_chunk, dch),
                                        jnp.bfloat16),
                   jax.ShapeDtypeStruct((nchunks, 1, 128), F32)],
    )(zf.reshape(nchunks, rows_per_chunk, dch), emb.astype(jnp.bfloat16))
    return zq.reshape(r, dch), jnp.sum(partial)


def _d2s(y4, h, w, cout):
    n = y4.shape[0]
    t = y4.reshape(n, h, w + 2, 2, 2, cout)[:, :, 1:w + 1]
    t = t.transpose(0, 1, 3, 2, 4, 5)
    return t.reshape(n, 2 * h, 2 * w, cout)


def kernel(x, w1, b1, w2, b2, w3, b3, emb_top, emb_bottom, wt, bt, wd1, bd1,
           wd2, bd2):
    n = x.shape[0]
    xh = jnp.transpose(x, (0, 2, 3, 1)).astype(jnp.bfloat16)

    y1 = _conv_layer(_s2d_pad_flat(xh), _s2_weight_mats(w1), b1,
                     112, 114, "relu")
    a1 = _unflat(y1, 112, 112)
    y2 = _conv_layer(_s2d_pad_flat(a1), _s2_weight_mats(w2), b2,
                     56, 58, "relu")
    zb = _unflat(y2, 56, 56)
    y3 = _conv_layer(_s2d_pad_flat(zb), _s2_weight_mats(w3), b3,
                     28, 30, "relu")
    zt = _unflat(y3, 28, 28)

    zt_q, ss_top = _vq(zt.reshape(n * 28 * 28, 128), emb_top, 3136)
    loss_top = 1.5 * ss_top / (n * 28 * 28 * 128)
    zt_q = zt_q.reshape(n, 28, 28, 128)

    y4 = _conv_layer(_pad_flat(zt_q), _ct_weight_mats_asplit(wt),
                     jnp.tile(bt, 4), 28, 30, "relu", mode="asplit")
    zt_up = _d2s(y4, 28, 28, 128)

    zb_q, ss_bot = _vq(zb.reshape(n * 56 * 56, 128), emb_bottom, 3136)
    loss_bottom = 1.5 * ss_bot / (n * 56 * 56 * 128)
    zb_q = zb_q.reshape(n, 56, 56, 128)

    zc = jnp.concatenate([zb_q, zt_up], axis=-1)
    y5 = _conv_layer(_pad_flat(zc), _ct_weight_mats_asplit(wd1),
                     jnp.tile(bd1, 4), 56, 58, "relu", mode="asplit")
    h1 = _d2s(y5, 56, 56, 64)

    y6 = _conv_layer(_pad_flat(h1), _ct_weight_mats(wd2),
                     jnp.tile(bd2, 4), 112, 114, "sigmoid", out_dtype=F32)
    x_hat = _d2s(y6, 112, 112, 3)
    x_hat = jnp.transpose(x_hat, (0, 3, 1, 2))

    return x_hat, loss_top + loss_bottom
